# Initial kernel scaffold; baseline (speedup 1.0000x reference)
import jax
import jax.numpy as jnp
from jax.experimental import pallas as pl


def kernel(input, targets):
    raise NotImplementedError("write your pallas kernel here")


# speedup vs baseline: 2.8073x; 2.8073x over previous
import functools

import jax
import jax.numpy as jnp
import numpy as np
from jax.experimental import pallas as pl
from jax.experimental.pallas import tpu as pltpu

_AW = (1.25, 2.0, 4.125, 3.75, 7.75, 7.375, 14.5, 19.5, 46.625)
_AH = (1.625, 3.75, 2.875, 7.625, 5.625, 14.875, 11.25, 24.75, 40.75)
_S = 52
_NT = 10
_NCLS = 80
_BS = 16
_EPS = 1e-7
_THR = 0.5


def _bce(p, t):
    p = jnp.clip(p, _EPS, 1.0 - _EPS)
    return -t * jnp.log(p) - (1.0 - t) * jnp.log(1.0 - p)


def _loss_body(tref, xref, oref):
    b = pl.program_id(0)
    a = pl.program_id(1)

    f32 = jnp.float32
    x_map = jax.nn.sigmoid(xref[0, 0])
    y_map = jax.nn.sigmoid(xref[0, 1])
    w_map = xref[0, 2]
    h_map = xref[0, 3]
    conf = jax.nn.sigmoid(xref[0, 4])

    ii = jax.lax.broadcasted_iota(jnp.int32, (_S, _S), 1).astype(f32)
    jj = jax.lax.broadcasted_iota(jnp.int32, (_S, _S), 0).astype(f32)

    aw_a = jnp.where(a == 0, _AW[6], jnp.where(a == 1, _AW[7], _AW[8]))
    ah_a = jnp.where(a == 0, _AH[6], jnp.where(a == 1, _AH[7], _AH[8]))

    pbx = x_map + ii
    pby = y_map + jj
    pbw = jnp.exp(w_map) * aw_a
    pbh = jnp.exp(h_map) * ah_a
    px1 = pbx - pbw * 0.5
    py1 = pby - pbh * 0.5
    px2 = pbx + pbw * 0.5
    py2 = pby + pbh * 0.5
    parea = (px2 - px1) * (py2 - py1)

    gxs, gys, gws, ghs, gi_f, gj_f, sel, cls_f = [], [], [], [], [], [], [], []
    for t in range(_NT):
        gx = tref[b, t, 0] * _S
        gy = tref[b, t, 1] * _S
        gw = tref[b, t, 2] * _S
        gh = tref[b, t, 3] * _S
        gi = jnp.floor(gx)
        gj = jnp.floor(gy)
        best_v = jnp.float32(-1.0)
        best_n = jnp.int32(0)
        for n in range(9):
            inter = (jnp.maximum(jnp.minimum(gw, _AW[n]), 0.0)
                     * jnp.maximum(jnp.minimum(gh, _AH[n]), 0.0))
            iou = inter / (gw * gh + _AW[n] * _AH[n] - inter + 1e-16)
            upd = iou > best_v
            best_v = jnp.where(upd, iou, best_v)
            best_n = jnp.where(upd, jnp.int32(n), best_n)
        ok = ((best_n == 6 + a) & (gj < _S) & (gi < _S)
              & (gi >= 0) & (gj >= 0))
        gxs.append(gx); gys.append(gy); gws.append(gw); ghs.append(gh)
        gi_f.append(jnp.clip(gi, 0.0, _S - 1.0))
        gj_f.append(jnp.clip(gj, 0.0, _S - 1.0))
        sel.append(ok)
        cls_f.append(jnp.clip(jnp.floor(tref[b, t, 4]), 0.0, _NCLS - 1.0))

    max_iou = jnp.zeros((_S, _S), f32)
    for t in range(_NT):
        gx1 = gxs[t] - gws[t] * 0.5
        gy1 = gys[t] - ghs[t] * 0.5
        gx2 = gxs[t] + gws[t] * 0.5
        gy2 = gys[t] + ghs[t] * 0.5
        iw = jnp.maximum(jnp.minimum(px2, gx2) - jnp.maximum(px1, gx1), 0.0)
        ih = jnp.maximum(jnp.minimum(py2, gy2) - jnp.maximum(py1, gy1), 0.0)
        inter = iw * ih
        garea = (gx2 - gx1) * (gy2 - gy1)
        iou = inter / (parea + garea - inter + 1e-16)
        max_iou = jnp.maximum(max_iou, iou)
    flag = (max_iou <= _THR).astype(f32)

    zmap = jnp.zeros((_S, _S), f32)
    mask_m = zmap
    tx_m = zmap; ty_m = zmap; tw_m = zmap; th_m = zmap
    bx_m = zmap; by_m = zmap
    for t in range(_NT):
        cm = (ii == gi_f[t]) & (jj == gj_f[t]) & sel[t]
        mask_m = jnp.where(cm, 1.0, mask_m)
        tx_m = jnp.where(cm, gxs[t] - gi_f[t], tx_m)
        ty_m = jnp.where(cm, gys[t] - gj_f[t], ty_m)
        tw_m = jnp.where(cm, jnp.log(jnp.maximum(gws[t], 1e-12) / aw_a), tw_m)
        th_m = jnp.where(cm, jnp.log(jnp.maximum(ghs[t], 1e-12) / ah_a), th_m)
        bx_m = jnp.where(cm, tref[b, t, 2], bx_m)
        by_m = jnp.where(cm, tref[b, t, 3], by_m)

    bls = (2.0 - bx_m) * (2.0 - by_m)
    noobj = (1.0 - mask_m) * flag

    lx = jnp.sum(_bce(x_map, tx_m) * bls * mask_m)
    ly = jnp.sum(_bce(y_map, ty_m) * bls * mask_m)
    lw = jnp.sum((w_map - tw_m) ** 2 * 0.5 * bls * mask_m)
    lh = jnp.sum((h_map - th_m) ** 2 * 0.5 * bls * mask_m)
    lconf = jnp.sum(_bce(conf, mask_m) * (mask_m + noobj))

    winner, rep = [], []
    for t in range(_NT):
        later_cell = False
        later_cc = False
        for t2 in range(t + 1, _NT):
            same_cell = sel[t2] & (gi_f[t2] == gi_f[t]) & (gj_f[t2] == gj_f[t])
            later_cell = later_cell | same_cell
            later_cc = later_cc | (same_cell & (cls_f[t2] == cls_f[t]))
        winner.append(sel[t] & (~later_cell))
        rep.append(sel[t] & (~later_cc))

    io80 = jax.lax.broadcasted_iota(jnp.int32, (1, _NCLS), 1).astype(f32)
    lcls = jnp.float32(0.0)
    for t in range(_NT):
        cm = ((ii == gi_f[t]) & (jj == gj_f[t]) & sel[t]).astype(f32)
        z = jnp.sum(xref[0, 5:85] * cm[None], axis=2)
        z = jnp.sum(z, axis=1).reshape(1, _NCLS)
        p = jnp.clip(jax.nn.sigmoid(z), _EPS, 1.0 - _EPS)
        sum0 = jnp.sum(-jnp.log(1.0 - p))
        oh = (io80 == cls_f[t]).astype(f32)
        pc = jnp.sum(p * oh)
        delta = -jnp.log(pc) + jnp.log(1.0 - pc)
        lcls = (lcls + jnp.where(winner[t], sum0, 0.0)
                + jnp.where(rep[t], delta, 0.0))

    @pl.when((b == 0) & (a == 0))
    def _init():
        for i in range(8):
            oref[i] = 0.0

    inv = 1.0 / _BS
    oref[1] += lx * inv
    oref[2] += ly * inv
    oref[3] += lw * inv
    oref[4] += lh * inv
    oref[5] += lconf * inv
    oref[6] += lcls * inv
    oref[0] += (lx + ly + lw + lh + lconf + lcls) * inv


@jax.jit
def kernel(input, targets):
    out = pl.pallas_call(
        _loss_body,
        grid=(_BS, 3),
        in_specs=[
            pl.BlockSpec(memory_space=pltpu.SMEM),
            pl.BlockSpec((1, 85, _S, _S), lambda b, a: (b, a, 0, 0)),
        ],
        out_specs=pl.BlockSpec(memory_space=pltpu.SMEM),
        out_shape=jax.ShapeDtypeStruct((8,), jnp.float32),
        compiler_params=pltpu.CompilerParams(
            dimension_semantics=("arbitrary", "arbitrary")),
    )(targets, input)
    return (out[0], out[1], out[2], out[3], out[4], out[5], out[6])


# speedup vs baseline: 9.3272x; 3.3224x over previous
import functools

import jax
import jax.numpy as jnp
import numpy as np
from jax import lax
from jax.experimental import pallas as pl
from jax.experimental.pallas import tpu as pltpu
from jax.experimental.pallas import tpu_sc as plsc

_AW = (1.25, 2.0, 4.125, 3.75, 7.75, 7.375, 14.5, 19.5, 46.625)
_AH = (1.625, 3.75, 2.875, 7.625, 5.625, 14.875, 11.25, 24.75, 40.75)
_S = 52
_NT = 10
_NCLS = 80
_BS = 16
_EPS = 1e-7
_THR = 0.5
_NW = 32
_NK = _BS * _NT
_NE = _NK * _NCLS
_EPW = _NE // _NW
_CPW = _EPW // 16


def _bce(p, t):
    p = jnp.clip(p, _EPS, 1.0 - _EPS)
    return -t * jnp.log(p) - (1.0 - t) * jnp.log(1.0 - p)


def _best_anchor(gw, gh):
    best_v = jnp.full_like(gw, -1.0)
    best_n = jnp.zeros(gw.shape, jnp.int32)
    for n in range(9):
        inter = (jnp.maximum(jnp.minimum(gw, _AW[n]), 0.0)
                 * jnp.maximum(jnp.minimum(gh, _AH[n]), 0.0))
        iou = inter / (gw * gh + _AW[n] * _AH[n] - inter + 1e-16)
        upd = iou > best_v
        best_v = jnp.where(upd, iou, best_v)
        best_n = jnp.where(upd, jnp.int32(n), best_n)
    return best_n


def _sc_body(inp_hbm, tgt_hbm, out_hbm, tv, chbv, gjv, giv,
             b0, b1, b2, b3, b4, outb, sem):
    lane = lax.iota(jnp.int32, 16)
    wid = lax.axis_index("s") * 2 + lax.axis_index("c")
    blks = [b0, b1, b2, b3, b4]

    pltpu.sync_copy(tgt_hbm, tv)

    for chunk in range(_NK // 16):
        k = chunk * 16 + lane
        b = k // _NT
        base5 = k * 5
        gx = plsc.load_gather(tv, [base5]) * float(_S)
        gy = plsc.load_gather(tv, [base5 + 1]) * float(_S)
        gw = plsc.load_gather(tv, [base5 + 2]) * float(_S)
        gh = plsc.load_gather(tv, [base5 + 3]) * float(_S)
        gi = gx.astype(jnp.int32)
        gj = gy.astype(jnp.int32)
        best_n = _best_anchor(gw, gh)
        valid = (best_n >= 6) & (gi < _S) & (gj < _S)
        bn = jnp.clip(best_n - 6, 0, 2)
        chbv[pl.ds(chunk * 16, 16)] = jnp.where(
            valid, b * 255 + bn * 85 + 5, 0)
        gjv[pl.ds(chunk * 16, 16)] = jnp.where(
            valid, jnp.clip(gj, 0, _S - 1), 0)
        giv[pl.ds(chunk * 16, 16)] = jnp.where(
            valid, jnp.clip(gi, 0, _S - 1), 0)

    copies = []
    for t in range(5):
        kvec = jnp.full((16,), wid * 5 + t, jnp.int32)
        chb = jnp.max(plsc.load_gather(chbv, [kvec]))
        gjs = jnp.max(plsc.load_gather(gjv, [kvec]))
        copies.append(pltpu.async_copy(
            inp_hbm.at[pl.ds(chb, _NCLS), pl.ds(gjs, 1)], blks[t], sem))
    for cp in copies:
        cp.wait()

    zero = jnp.zeros((16,), jnp.int32)
    for t in range(5):
        kvec = jnp.full((16,), wid * 5 + t, jnp.int32)
        gvec = plsc.load_gather(giv, [kvec])
        for chunk in range(5):
            c = chunk * 16 + lane
            outb[pl.ds(t * _NCLS + chunk * 16, 16)] = plsc.load_gather(
                blks[t], [c, zero, gvec])

    pltpu.sync_copy(outb, out_hbm.at[pl.ds(wid * _EPW, _EPW)])


_sc_gather = functools.partial(
    pl.kernel,
    mesh=plsc.VectorSubcoreMesh(core_axis_name="c", subcore_axis_name="s"),
    out_type=jax.ShapeDtypeStruct((_NE,), jnp.float32),
    scratch_types=(
        [pltpu.VMEM((_BS * _NT * 5,), jnp.float32),
         pltpu.VMEM((_NK,), jnp.int32),
         pltpu.VMEM((_NK,), jnp.int32),
         pltpu.VMEM((_NK,), jnp.int32)]
        + [pltpu.VMEM((_NCLS, 1, _S), jnp.float32) for _ in range(5)]
        + [pltpu.VMEM((_EPW,), jnp.float32),
           pltpu.SemaphoreType.DMA]
    ),
    compiler_params=pltpu.CompilerParams(needs_layout_passes=False),
)(_sc_body)


def _loss_body(tref, xref, cref, oref):
    b = pl.program_id(0)
    a = pl.program_id(1)

    f32 = jnp.float32
    x_map = jax.nn.sigmoid(xref[0, 0])
    y_map = jax.nn.sigmoid(xref[0, 1])
    w_map = xref[0, 2]
    h_map = xref[0, 3]
    conf = jax.nn.sigmoid(xref[0, 4])

    ii = jax.lax.broadcasted_iota(jnp.int32, (_S, _S), 1).astype(f32)
    jj = jax.lax.broadcasted_iota(jnp.int32, (_S, _S), 0).astype(f32)

    aw_a = jnp.where(a == 0, _AW[6], jnp.where(a == 1, _AW[7], _AW[8]))
    ah_a = jnp.where(a == 0, _AH[6], jnp.where(a == 1, _AH[7], _AH[8]))

    pbx = x_map + ii
    pby = y_map + jj
    pbw = jnp.exp(w_map) * aw_a
    pbh = jnp.exp(h_map) * ah_a
    px1 = pbx - pbw * 0.5
    py1 = pby - pbh * 0.5
    px2 = pbx + pbw * 0.5
    py2 = pby + pbh * 0.5
    parea = (px2 - px1) * (py2 - py1)

    gxs, gys, gws, ghs, gi_f, gj_f, sel, cls_f = [], [], [], [], [], [], [], []
    for t in range(_NT):
        gx = tref[b, t, 0] * _S
        gy = tref[b, t, 1] * _S
        gw = tref[b, t, 2] * _S
        gh = tref[b, t, 3] * _S
        gi = jnp.floor(gx)
        gj = jnp.floor(gy)
        best_n = _best_anchor(gw, gh)
        ok = ((best_n == 6 + a) & (gj < _S) & (gi < _S)
              & (gi >= 0) & (gj >= 0))
        gxs.append(gx); gys.append(gy); gws.append(gw); ghs.append(gh)
        gi_f.append(jnp.clip(gi, 0.0, _S - 1.0))
        gj_f.append(jnp.clip(gj, 0.0, _S - 1.0))
        sel.append(ok)
        cls_f.append(jnp.clip(jnp.floor(tref[b, t, 4]), 0.0, _NCLS - 1.0))

    max_iou = jnp.zeros((_S, _S), f32)
    for t in range(_NT):
        gx1 = gxs[t] - gws[t] * 0.5
        gy1 = gys[t] - ghs[t] * 0.5
        gx2 = gxs[t] + gws[t] * 0.5
        gy2 = gys[t] + ghs[t] * 0.5
        iw = jnp.maximum(jnp.minimum(px2, gx2) - jnp.maximum(px1, gx1), 0.0)
        ih = jnp.maximum(jnp.minimum(py2, gy2) - jnp.maximum(py1, gy1), 0.0)
        inter = iw * ih
        garea = (gx2 - gx1) * (gy2 - gy1)
        iou = inter / (parea + garea - inter + 1e-16)
        max_iou = jnp.maximum(max_iou, iou)
    flag = (max_iou <= _THR).astype(f32)

    zmap = jnp.zeros((_S, _S), f32)
    mask_m = zmap
    tx_m = zmap; ty_m = zmap; tw_m = zmap; th_m = zmap
    bx_m = zmap; by_m = zmap
    for t in range(_NT):
        cm = (ii == gi_f[t]) & (jj == gj_f[t]) & sel[t]
        mask_m = jnp.where(cm, 1.0, mask_m)
        tx_m = jnp.where(cm, gxs[t] - gi_f[t], tx_m)
        ty_m = jnp.where(cm, gys[t] - gj_f[t], ty_m)
        tw_m = jnp.where(cm, jnp.log(jnp.maximum(gws[t], 1e-12) / aw_a), tw_m)
        th_m = jnp.where(cm, jnp.log(jnp.maximum(ghs[t], 1e-12) / ah_a), th_m)
        bx_m = jnp.where(cm, tref[b, t, 2], bx_m)
        by_m = jnp.where(cm, tref[b, t, 3], by_m)

    bls = (2.0 - bx_m) * (2.0 - by_m)
    noobj = (1.0 - mask_m) * flag

    lx = jnp.sum(_bce(x_map, tx_m) * bls * mask_m)
    ly = jnp.sum(_bce(y_map, ty_m) * bls * mask_m)
    lw = jnp.sum((w_map - tw_m) ** 2 * 0.5 * bls * mask_m)
    lh = jnp.sum((h_map - th_m) ** 2 * 0.5 * bls * mask_m)
    lconf = jnp.sum(_bce(conf, mask_m) * (mask_m + noobj))

    winner, rep = [], []
    for t in range(_NT):
        later_cell = False
        later_cc = False
        for t2 in range(t + 1, _NT):
            same_cell = sel[t2] & (gi_f[t2] == gi_f[t]) & (gj_f[t2] == gj_f[t])
            later_cell = later_cell | same_cell
            later_cc = later_cc | (same_cell & (cls_f[t2] == cls_f[t]))
        winner.append(sel[t] & (~later_cell))
        rep.append(sel[t] & (~later_cc))

    io80 = jax.lax.broadcasted_iota(jnp.int32, (1, _NCLS), 1).astype(f32)
    lcls = jnp.float32(0.0)
    for t in range(_NT):
        z = cref[0, t].reshape(1, _NCLS)
        p = jnp.clip(jax.nn.sigmoid(z), _EPS, 1.0 - _EPS)
        sum0 = jnp.sum(-jnp.log(1.0 - p))
        oh = (io80 == cls_f[t]).astype(f32)
        pc = jnp.sum(p * oh)
        delta = -jnp.log(pc) + jnp.log(1.0 - pc)
        lcls = (lcls + jnp.where(winner[t], sum0, 0.0)
                + jnp.where(rep[t], delta, 0.0))

    @pl.when((b == 0) & (a == 0))
    def _init():
        for i in range(8):
            oref[i] = 0.0

    inv = 1.0 / _BS
    oref[1] += lx * inv
    oref[2] += ly * inv
    oref[3] += lw * inv
    oref[4] += lh * inv
    oref[5] += lconf * inv
    oref[6] += lcls * inv
    oref[0] += (lx + ly + lw + lh + lconf + lcls) * inv


@jax.jit
def kernel(input, targets):
    cls = _sc_gather(input.reshape(_BS * 255, _S, _S),
                     targets.reshape(_BS * _NT * 5))
    cls = cls.reshape(_BS, _NT, _NCLS)
    out = pl.pallas_call(
        _loss_body,
        grid=(_BS, 3),
        in_specs=[
            pl.BlockSpec(memory_space=pltpu.SMEM),
            pl.BlockSpec((1, 5, _S, _S), lambda b, a: (b, 17 * a, 0, 0)),
            pl.BlockSpec((1, _NT, _NCLS), lambda b, a: (b, 0, 0)),
        ],
        out_specs=pl.BlockSpec(memory_space=pltpu.SMEM),
        out_shape=jax.ShapeDtypeStruct((8,), jnp.float32),
        compiler_params=pltpu.CompilerParams(
            dimension_semantics=("arbitrary", "arbitrary")),
    )(targets, input, cls)
    return (out[0], out[1], out[2], out[3], out[4], out[5], out[6])


# speedup vs baseline: 11.2398x; 1.2051x over previous
import functools

import jax
import jax.numpy as jnp
import numpy as np
from jax import lax
from jax.experimental import pallas as pl
from jax.experimental.pallas import tpu as pltpu
from jax.experimental.pallas import tpu_sc as plsc

_AW = (1.25, 2.0, 4.125, 3.75, 7.75, 7.375, 14.5, 19.5, 46.625)
_AH = (1.625, 3.75, 2.875, 7.625, 5.625, 14.875, 11.25, 24.75, 40.75)
_S = 52
_NT = 10
_NCLS = 80
_BS = 16
_EPS = 1e-7
_THR = 0.5
_NW = 32
_NK = _BS * _NT
_NE = _NK * _NCLS
_EPW = _NE // _NW
_CPW = _EPW // 16


def _bce(p, t):
    p = jnp.clip(p, _EPS, 1.0 - _EPS)
    return -t * jnp.log(p) - (1.0 - t) * jnp.log(1.0 - p)


def _best_anchor(gw, gh):
    best_v = jnp.full_like(gw, -1.0)
    best_n = jnp.zeros(gw.shape, jnp.int32)
    for n in range(9):
        inter = (jnp.maximum(jnp.minimum(gw, _AW[n]), 0.0)
                 * jnp.maximum(jnp.minimum(gh, _AH[n]), 0.0))
        iou = inter / (gw * gh + _AW[n] * _AH[n] - inter + 1e-16)
        upd = iou > best_v
        best_v = jnp.where(upd, iou, best_v)
        best_n = jnp.where(upd, jnp.int32(n), best_n)
    return best_n


def _sc_body(inp_hbm, tgt_hbm, out_hbm, tv, chbv, gjv, giv,
             b0, b1, b2, b3, b4, outb, sem):
    lane = lax.iota(jnp.int32, 16)
    wid = lax.axis_index("s") * 2 + lax.axis_index("c")
    blks = [b0, b1, b2, b3, b4]

    pltpu.sync_copy(tgt_hbm, tv)

    for chunk in range(_NK // 16):
        k = chunk * 16 + lane
        b = k // _NT
        base5 = k * 5
        gx = plsc.load_gather(tv, [base5]) * float(_S)
        gy = plsc.load_gather(tv, [base5 + 1]) * float(_S)
        gw = plsc.load_gather(tv, [base5 + 2]) * float(_S)
        gh = plsc.load_gather(tv, [base5 + 3]) * float(_S)
        gi = gx.astype(jnp.int32)
        gj = gy.astype(jnp.int32)
        best_n = _best_anchor(gw, gh)
        valid = (best_n >= 6) & (gi < _S) & (gj < _S)
        bn = jnp.clip(best_n - 6, 0, 2)
        chbv[pl.ds(chunk * 16, 16)] = jnp.where(
            valid, b * 255 + bn * 85 + 5, 0)
        gjv[pl.ds(chunk * 16, 16)] = jnp.where(
            valid, jnp.clip(gj, 0, _S - 1), 0)
        giv[pl.ds(chunk * 16, 16)] = jnp.where(
            valid, jnp.clip(gi, 0, _S - 1), 0)

    copies = []
    for t in range(5):
        kvec = jnp.full((16,), wid * 5 + t, jnp.int32)
        chb = jnp.max(plsc.load_gather(chbv, [kvec]))
        gjs = jnp.max(plsc.load_gather(gjv, [kvec]))
        copies.append(pltpu.async_copy(
            inp_hbm.at[pl.ds(chb, _NCLS), pl.ds(gjs, 1)], blks[t], sem))
    for cp in copies:
        cp.wait()

    zero = jnp.zeros((16,), jnp.int32)
    for t in range(5):
        kvec = jnp.full((16,), wid * 5 + t, jnp.int32)
        gvec = plsc.load_gather(giv, [kvec])
        for chunk in range(5):
            c = chunk * 16 + lane
            outb[pl.ds(t * _NCLS + chunk * 16, 16)] = plsc.load_gather(
                blks[t], [c, zero, gvec])

    pltpu.sync_copy(outb, out_hbm.at[pl.ds(wid * _EPW, _EPW)])


_sc_gather = functools.partial(
    pl.kernel,
    mesh=plsc.VectorSubcoreMesh(core_axis_name="c", subcore_axis_name="s"),
    out_type=jax.ShapeDtypeStruct((_NE,), jnp.float32),
    scratch_types=(
        [pltpu.VMEM((_BS * _NT * 5,), jnp.float32),
         pltpu.VMEM((_NK,), jnp.int32),
         pltpu.VMEM((_NK,), jnp.int32),
         pltpu.VMEM((_NK,), jnp.int32)]
        + [pltpu.VMEM((_NCLS, 1, _S), jnp.float32) for _ in range(5)]
        + [pltpu.VMEM((_EPW,), jnp.float32),
           pltpu.SemaphoreType.DMA]
    ),
    compiler_params=pltpu.CompilerParams(needs_layout_passes=False),
)(_sc_body)


def _loss_body(tref, x0ref, x1ref, x2ref, cref, oref):
    b = pl.program_id(0)
    xrefs = (x0ref, x1ref, x2ref)

    f32 = jnp.float32
    ii = jax.lax.broadcasted_iota(jnp.int32, (_S, _S), 1).astype(f32)
    jj = jax.lax.broadcasted_iota(jnp.int32, (_S, _S), 0).astype(f32)

    gxs, gys, gws, ghs, gi_f, gj_f, cls_f = [], [], [], [], [], [], []
    sel_any, bns, tws, ths = [], [], [], []
    for t in range(_NT):
        gx = tref[b, t, 0] * _S
        gy = tref[b, t, 1] * _S
        gw = tref[b, t, 2] * _S
        gh = tref[b, t, 3] * _S
        gi = jnp.floor(gx)
        gj = jnp.floor(gy)
        best_n = _best_anchor(gw, gh)
        ok = ((best_n >= 6) & (gj < _S) & (gi < _S)
              & (gi >= 0) & (gj >= 0))
        bn = jnp.clip(best_n - 6, 0, 2)
        aw_t = jnp.where(bn == 0, _AW[6], jnp.where(bn == 1, _AW[7], _AW[8]))
        ah_t = jnp.where(bn == 0, _AH[6], jnp.where(bn == 1, _AH[7], _AH[8]))
        gxs.append(gx); gys.append(gy); gws.append(gw); ghs.append(gh)
        gi_f.append(jnp.clip(gi, 0.0, _S - 1.0))
        gj_f.append(jnp.clip(gj, 0.0, _S - 1.0))
        sel_any.append(ok)
        bns.append(bn)
        tws.append(jnp.log(jnp.maximum(gw, 1e-12) / aw_t))
        ths.append(jnp.log(jnp.maximum(gh, 1e-12) / ah_t))
        cls_f.append(jnp.clip(jnp.floor(tref[b, t, 4]), 0.0, _NCLS - 1.0))

    winner, rep = [], []
    for t in range(_NT):
        later_cell = False
        later_cc = False
        for t2 in range(t + 1, _NT):
            same_cell = (sel_any[t2] & (bns[t2] == bns[t])
                         & (gi_f[t2] == gi_f[t]) & (gj_f[t2] == gj_f[t]))
            later_cell = later_cell | same_cell
            later_cc = later_cc | (same_cell & (cls_f[t2] == cls_f[t]))
        winner.append(sel_any[t] & (~later_cell))
        rep.append(sel_any[t] & (~later_cc))

    io80 = jax.lax.broadcasted_iota(jnp.int32, (1, _NCLS), 1).astype(f32)
    lcls = jnp.float32(0.0)
    for t in range(_NT):
        z = cref[0, t].reshape(1, _NCLS)
        p = jnp.clip(jax.nn.sigmoid(z), _EPS, 1.0 - _EPS)
        sum0 = jnp.sum(-jnp.log(1.0 - p))
        oh = (io80 == cls_f[t]).astype(f32)
        pc = jnp.sum(p * oh)
        delta = -jnp.log(pc) + jnp.log(1.0 - pc)
        lcls = (lcls + jnp.where(winner[t], sum0, 0.0)
                + jnp.where(rep[t], delta, 0.0))

    lx = jnp.float32(0.0); ly = jnp.float32(0.0)
    lw = jnp.float32(0.0); lh = jnp.float32(0.0)
    lconf = jnp.float32(0.0)
    for a in range(3):
        x_map = jax.nn.sigmoid(xrefs[a][0, 0])
        y_map = jax.nn.sigmoid(xrefs[a][0, 1])
        w_map = xrefs[a][0, 2]
        h_map = xrefs[a][0, 3]
        conf = jax.nn.sigmoid(xrefs[a][0, 4])

        pbx = x_map + ii
        pby = y_map + jj
        pbw = jnp.exp(w_map) * _AW[6 + a]
        pbh = jnp.exp(h_map) * _AH[6 + a]
        px1 = pbx - pbw * 0.5
        py1 = pby - pbh * 0.5
        px2 = pbx + pbw * 0.5
        py2 = pby + pbh * 0.5
        parea = (px2 - px1) * (py2 - py1)

        max_iou = jnp.zeros((_S, _S), f32)
        for t in range(_NT):
            gx1 = gxs[t] - gws[t] * 0.5
            gy1 = gys[t] - ghs[t] * 0.5
            gx2 = gxs[t] + gws[t] * 0.5
            gy2 = gys[t] + ghs[t] * 0.5
            iw = jnp.maximum(
                jnp.minimum(px2, gx2) - jnp.maximum(px1, gx1), 0.0)
            ih = jnp.maximum(
                jnp.minimum(py2, gy2) - jnp.maximum(py1, gy1), 0.0)
            inter = iw * ih
            garea = (gx2 - gx1) * (gy2 - gy1)
            iou = inter / (parea + garea - inter + 1e-16)
            max_iou = jnp.maximum(max_iou, iou)
        flag = (max_iou <= _THR).astype(f32)

        zmap = jnp.zeros((_S, _S), f32)
        mask_m = zmap
        tx_m = zmap; ty_m = zmap; tw_m = zmap; th_m = zmap
        bx_m = zmap; by_m = zmap
        for t in range(_NT):
            cm = ((ii == gi_f[t]) & (jj == gj_f[t])
                  & sel_any[t] & (bns[t] == a))
            mask_m = jnp.where(cm, 1.0, mask_m)
            tx_m = jnp.where(cm, gxs[t] - gi_f[t], tx_m)
            ty_m = jnp.where(cm, gys[t] - gj_f[t], ty_m)
            tw_m = jnp.where(cm, tws[t], tw_m)
            th_m = jnp.where(cm, ths[t], th_m)
            bx_m = jnp.where(cm, tref[b, t, 2], bx_m)
            by_m = jnp.where(cm, tref[b, t, 3], by_m)

        bls = (2.0 - bx_m) * (2.0 - by_m)
        noobj = (1.0 - mask_m) * flag

        lx += jnp.sum(_bce(x_map, tx_m) * bls * mask_m)
        ly += jnp.sum(_bce(y_map, ty_m) * bls * mask_m)
        lw += jnp.sum((w_map - tw_m) ** 2 * 0.5 * bls * mask_m)
        lh += jnp.sum((h_map - th_m) ** 2 * 0.5 * bls * mask_m)
        lconf += jnp.sum(_bce(conf, mask_m) * (mask_m + noobj))

    @pl.when(b == 0)
    def _init():
        for i in range(8):
            oref[i] = 0.0

    inv = 1.0 / _BS
    oref[1] += lx * inv
    oref[2] += ly * inv
    oref[3] += lw * inv
    oref[4] += lh * inv
    oref[5] += lconf * inv
    oref[6] += lcls * inv
    oref[0] += (lx + ly + lw + lh + lconf + lcls) * inv


@jax.jit
def kernel(input, targets):
    cls = _sc_gather(input.reshape(_BS * 255, _S, _S),
                     targets.reshape(_BS * _NT * 5))
    cls = cls.reshape(_BS, _NT, _NCLS)
    out = pl.pallas_call(
        _loss_body,
        grid=(_BS,),
        in_specs=[
            pl.BlockSpec(memory_space=pltpu.SMEM),
            pl.BlockSpec((1, 5, _S, _S), lambda b: (b, 0, 0, 0)),
            pl.BlockSpec((1, 5, _S, _S), lambda b: (b, 17, 0, 0)),
            pl.BlockSpec((1, 5, _S, _S), lambda b: (b, 34, 0, 0)),
            pl.BlockSpec((1, _NT, _NCLS), lambda b: (b, 0, 0)),
        ],
        out_specs=pl.BlockSpec(memory_space=pltpu.SMEM),
        out_shape=jax.ShapeDtypeStruct((8,), jnp.float32),
        compiler_params=pltpu.CompilerParams(
            dimension_semantics=("arbitrary",)),
    )(targets, input, input, input, cls)
    return (out[0], out[1], out[2], out[3], out[4], out[5], out[6])
